# baseline (speedup 1.0000x reference)
import functools

import jax
import jax.numpy as jnp
from jax import lax
from jax.experimental import pallas as pl
from jax.experimental.pallas import tpu as pltpu
from jax.experimental.pallas import tpu_sc as plsc

VOCAB = 1000000
EMBED_DIM = 64
BATCH = 16384
HIST = 50

B = BATCH * HIST
NC, NS = 2, 16
NW = NC * NS
B_PER_W = B // NW
IDX_MINOR = 128
N_SUB = 5
CHUNK = N_SUB * IDX_MINOR
N_ITERS = B_PER_W // CHUNK
IDX_ROWS_PER_CHUNK = CHUNK // IDX_MINOR


def _make_kernel():
    mesh = plsc.VectorSubcoreMesh(core_axis_name="c", subcore_axis_name="s")

    @functools.partial(
        pl.kernel,
        mesh=mesh,
        out_type=jax.ShapeDtypeStruct((B, EMBED_DIM), jnp.float32),
        compiler_params=pltpu.CompilerParams(use_tc_tiling_on_sc=False),
        scratch_types=[
            pltpu.VMEM((2, N_SUB, IDX_MINOR), jnp.int32),
            pltpu.VMEM((2, CHUNK, EMBED_DIM), jnp.float32),
            pltpu.SemaphoreType.DMA,
            pltpu.SemaphoreType.DMA,
            pltpu.SemaphoreType.DMA,
            pltpu.SemaphoreType.DMA,
            pltpu.SemaphoreType.DMA,
            pltpu.SemaphoreType.DMA,
        ],
    )
    def k(idx_hbm, table_hbm, out_hbm, idx_v, rows_v,
          idx_sem0, idx_sem1, gat_sem0, gat_sem1, out_sem0, out_sem1):
        idx_sems = (idx_sem0, idx_sem1)
        gat_sems = (gat_sem0, gat_sem1)
        out_sems = (out_sem0, out_sem1)

        wid = lax.axis_index("s") * NC + lax.axis_index("c")
        base = wid * B_PER_W
        base128 = wid * (B_PER_W // IDX_MINOR)

        def idx_copy(i, b):
            return pltpu.make_async_copy(
                idx_hbm.at[pl.ds(base128 + i * N_SUB, N_SUB)],
                idx_v.at[b],
                idx_sems[b],
            )

        def fire_gathers(b):
            for j in range(N_SUB):
                pltpu.async_copy(
                    table_hbm.at[idx_v.at[b].at[j]],
                    rows_v.at[b].at[pl.ds(j * IDX_MINOR, IDX_MINOR)],
                    gat_sems[b],
                )

        def wait_gathers(b):
            for j in range(N_SUB):
                pltpu.make_async_copy(
                    table_hbm.at[idx_v.at[b].at[j]],
                    rows_v.at[b].at[pl.ds(j * IDX_MINOR, IDX_MINOR)],
                    gat_sems[b],
                ).wait()

        def store_copy(i, b):
            return pltpu.make_async_copy(
                rows_v.at[b],
                out_hbm.at[pl.ds(base + i * CHUNK, CHUNK)],
                out_sems[b],
            )

        idx_copy(0, 0).start()
        idx_copy(1, 1).start()
        idx_copy(0, 0).wait()
        fire_gathers(0)

        def body(g, _):
            for b in range(2):
                i = 2 * g + b
                @pl.when(i + 1 < N_ITERS)
                def _():
                    idx_copy(i + 1, 1 - b).wait()

                    @pl.when(i >= 1)
                    def _():
                        store_copy(i - 1, 1 - b).wait()

                    fire_gathers(1 - b)

                wait_gathers(b)
                @pl.when(i + 2 < N_ITERS)
                def _():
                    idx_copy(i + 2, b).start()
                store_copy(i, b).start()
            return 0

        lax.fori_loop(0, N_ITERS // 2, body, 0)

        store_copy(N_ITERS - 2, 0).wait()
        store_copy(N_ITERS - 1, 1).wait()

    return k


_gather = _make_kernel()


def kernel(inputs, table):
    idx2d = inputs.reshape(B // IDX_MINOR, IDX_MINOR)
    out = _gather(idx2d, table)
    return out.reshape(BATCH, HIST, EMBED_DIM)


# speedup vs baseline: 1.0009x; 1.0009x over previous
import functools

import jax
import jax.numpy as jnp
from jax import lax
from jax.experimental import pallas as pl
from jax.experimental.pallas import tpu as pltpu
from jax.experimental.pallas import tpu_sc as plsc

VOCAB = 1000000
EMBED_DIM = 64
BATCH = 16384
HIST = 50

B = BATCH * HIST
NC, NS = 2, 16
NW = NC * NS
B_PER_W = B // NW
IDX_MINOR = 128
N_SUB = 4
CHUNK = N_SUB * IDX_MINOR
N_ITERS = B_PER_W // CHUNK
IDX_ROWS_PER_CHUNK = CHUNK // IDX_MINOR


def _make_kernel():
    mesh = plsc.VectorSubcoreMesh(core_axis_name="c", subcore_axis_name="s")

    @functools.partial(
        pl.kernel,
        mesh=mesh,
        out_type=jax.ShapeDtypeStruct((B, EMBED_DIM), jnp.float32),
        compiler_params=pltpu.CompilerParams(use_tc_tiling_on_sc=False),
        scratch_types=[
            pltpu.VMEM((2, N_SUB, IDX_MINOR), jnp.int32),
            pltpu.VMEM((2, CHUNK, EMBED_DIM), jnp.float32),
            pltpu.SemaphoreType.DMA,
            pltpu.SemaphoreType.DMA,
            pltpu.SemaphoreType.DMA,
            pltpu.SemaphoreType.DMA,
            pltpu.SemaphoreType.DMA,
            pltpu.SemaphoreType.DMA,
        ],
    )
    def k(idx_hbm, table_hbm, out_hbm, idx_v, rows_v,
          idx_sem0, idx_sem1, gat_sem0, gat_sem1, out_sem0, out_sem1):
        idx_sems = (idx_sem0, idx_sem1)
        gat_sems = (gat_sem0, gat_sem1)
        out_sems = (out_sem0, out_sem1)

        wid = lax.axis_index("s") * NC + lax.axis_index("c")
        base = wid * B_PER_W
        base128 = wid * (B_PER_W // IDX_MINOR)

        def idx_copy(i, b):
            return pltpu.make_async_copy(
                idx_hbm.at[pl.ds(base128 + i * N_SUB, N_SUB)],
                idx_v.at[b],
                idx_sems[b],
            )

        def fire_gathers(b):
            for j in range(N_SUB):
                pltpu.async_copy(
                    table_hbm.at[idx_v.at[b].at[j]],
                    rows_v.at[b].at[pl.ds(j * IDX_MINOR, IDX_MINOR)],
                    gat_sems[b],
                )

        def wait_gathers(b):
            for j in range(N_SUB):
                pltpu.make_async_copy(
                    table_hbm.at[idx_v.at[b].at[j]],
                    rows_v.at[b].at[pl.ds(j * IDX_MINOR, IDX_MINOR)],
                    gat_sems[b],
                ).wait()

        def store_copy(i, b):
            return pltpu.make_async_copy(
                rows_v.at[b],
                out_hbm.at[pl.ds(base + i * CHUNK, CHUNK)],
                out_sems[b],
            )

        idx_copy(0, 0).start()
        idx_copy(1, 1).start()
        idx_copy(0, 0).wait()
        fire_gathers(0)

        def body(g, _):
            for b in range(2):
                i = 2 * g + b
                @pl.when(i + 1 < N_ITERS)
                def _():
                    idx_copy(i + 1, 1 - b).wait()

                    @pl.when(i >= 1)
                    def _():
                        store_copy(i - 1, 1 - b).wait()

                    fire_gathers(1 - b)

                wait_gathers(b)
                @pl.when(i + 2 < N_ITERS)
                def _():
                    idx_copy(i + 2, b).start()
                store_copy(i, b).start()
            return 0

        lax.fori_loop(0, N_ITERS // 2, body, 0)

        store_copy(N_ITERS - 2, 0).wait()
        store_copy(N_ITERS - 1, 1).wait()

    return k


_gather = _make_kernel()


def kernel(inputs, table):
    idx2d = inputs.reshape(B // IDX_MINOR, IDX_MINOR)
    out = _gather(idx2d, table)
    return out.reshape(BATCH, HIST, EMBED_DIM)
